# baseline (speedup 1.0000x reference)
import jax
import jax.numpy as jnp
from jax.experimental import pallas as pl
from jax.experimental.pallas import tpu as pltpu

ROW_BLK = 128
N_ROW = 4096
N_COL = 2048
N_BATCH = 4
N_K = N_ROW // ROW_BLK


def _argmin_body(x_ref, o_ref, mval, midx):
    k = pl.program_id(1)
    xb = x_ref[0]
    bm = jnp.min(xb, axis=0, keepdims=True)
    rows = jax.lax.broadcasted_iota(jnp.int32, (ROW_BLK, N_COL), 0) + k * ROW_BLK
    bi = jnp.min(jnp.where(xb == bm, rows, jnp.int32(2**30)), axis=0, keepdims=True)

    @pl.when(k == 0)
    def _init():
        mval[...] = bm
        midx[...] = bi

    @pl.when(k > 0)
    def _merge():
        better = bm < mval[...]
        mval[...] = jnp.where(better, bm, mval[...])
        midx[...] = jnp.where(better, bi, midx[...])

    @pl.when(k == N_K - 1)
    def _emit():
        o_ref[0] = midx[...]


def kernel(x):
    out = pl.pallas_call(
        _argmin_body,
        grid=(N_BATCH, N_K),
        in_specs=[pl.BlockSpec((1, ROW_BLK, N_COL), lambda b, k: (b, k, 0))],
        out_specs=pl.BlockSpec((1, 1, N_COL), lambda b, k: (b, 0, 0)),
        out_shape=jax.ShapeDtypeStruct((N_BATCH, 1, N_COL), jnp.int32),
        scratch_shapes=[
            pltpu.VMEM((1, N_COL), jnp.float32),
            pltpu.VMEM((1, N_COL), jnp.int32),
        ],
    )(x)
    return out.reshape(N_BATCH, N_COL).astype(jnp.int64)


# speedup vs baseline: 1.9517x; 1.9517x over previous
import jax
import jax.numpy as jnp
from jax.experimental import pallas as pl
from jax.experimental.pallas import tpu as pltpu

ROW_BLK = 512
N_ROW = 4096
N_COL = 2048
N_BATCH = 4
N_K = N_ROW // ROW_BLK


def _argmin_body(x_ref, o_ref, mval, midx):
    k = pl.program_id(1)
    xb = x_ref[0]
    bm = jnp.min(xb, axis=0, keepdims=True)
    rows = jax.lax.broadcasted_iota(jnp.int32, (ROW_BLK, N_COL), 0) + k * ROW_BLK
    bi = jnp.min(jnp.where(xb == bm, rows, jnp.int32(2**30)), axis=0, keepdims=True)

    @pl.when(k == 0)
    def _init():
        mval[...] = bm
        midx[...] = bi

    @pl.when(k > 0)
    def _merge():
        better = bm < mval[...]
        mval[...] = jnp.where(better, bm, mval[...])
        midx[...] = jnp.where(better, bi, midx[...])

    @pl.when(k == N_K - 1)
    def _emit():
        o_ref[0] = midx[...]


def kernel(x):
    out = pl.pallas_call(
        _argmin_body,
        grid=(N_BATCH, N_K),
        in_specs=[pl.BlockSpec((1, ROW_BLK, N_COL), lambda b, k: (b, k, 0))],
        out_specs=pl.BlockSpec((1, 1, N_COL), lambda b, k: (b, 0, 0)),
        out_shape=jax.ShapeDtypeStruct((N_BATCH, 1, N_COL), jnp.int32),
        scratch_shapes=[
            pltpu.VMEM((1, N_COL), jnp.float32),
            pltpu.VMEM((1, N_COL), jnp.int32),
        ],
    )(x)
    return out.reshape(N_BATCH, N_COL).astype(jnp.int64)


# speedup vs baseline: 2.2607x; 1.1583x over previous
import jax
import jax.numpy as jnp
from jax.experimental import pallas as pl
from jax.experimental.pallas import tpu as pltpu

ROW_BLK = 1024
N_ROW = 4096
N_COL = 2048
N_BATCH = 4
N_K = N_ROW // ROW_BLK


def _argmin_body(x_ref, o_ref, mval, midx):
    k = pl.program_id(1)
    xb = x_ref[0]
    bm = jnp.min(xb, axis=0, keepdims=True)
    rows = jax.lax.broadcasted_iota(jnp.int32, (ROW_BLK, N_COL), 0) + k * ROW_BLK
    bi = jnp.min(jnp.where(xb == bm, rows, jnp.int32(2**30)), axis=0, keepdims=True)

    @pl.when(k == 0)
    def _init():
        mval[...] = bm
        midx[...] = bi

    @pl.when(k > 0)
    def _merge():
        better = bm < mval[...]
        mval[...] = jnp.where(better, bm, mval[...])
        midx[...] = jnp.where(better, bi, midx[...])

    @pl.when(k == N_K - 1)
    def _emit():
        o_ref[0] = midx[...]


def kernel(x):
    out = pl.pallas_call(
        _argmin_body,
        grid=(N_BATCH, N_K),
        in_specs=[pl.BlockSpec((1, ROW_BLK, N_COL), lambda b, k: (b, k, 0))],
        out_specs=pl.BlockSpec((1, 1, N_COL), lambda b, k: (b, 0, 0)),
        out_shape=jax.ShapeDtypeStruct((N_BATCH, 1, N_COL), jnp.int32),
        scratch_shapes=[
            pltpu.VMEM((1, N_COL), jnp.float32),
            pltpu.VMEM((1, N_COL), jnp.int32),
        ],
    )(x)
    return out.reshape(N_BATCH, N_COL).astype(jnp.int64)


# speedup vs baseline: 2.3257x; 1.0287x over previous
import jax
import jax.numpy as jnp
from jax.experimental import pallas as pl
from jax.experimental.pallas import tpu as pltpu

ROW_BLK = 2048
N_ROW = 4096
N_COL = 2048
N_BATCH = 4
N_K = N_ROW // ROW_BLK


def _argmin_body(x_ref, o_ref, mval, midx):
    k = pl.program_id(1)
    xb = x_ref[0]
    bm = jnp.min(xb, axis=0, keepdims=True)
    rows = jax.lax.broadcasted_iota(jnp.int32, (ROW_BLK, N_COL), 0) + k * ROW_BLK
    bi = jnp.min(jnp.where(xb == bm, rows, jnp.int32(2**30)), axis=0, keepdims=True)

    @pl.when(k == 0)
    def _init():
        mval[...] = bm
        midx[...] = bi

    @pl.when(k > 0)
    def _merge():
        better = bm < mval[...]
        mval[...] = jnp.where(better, bm, mval[...])
        midx[...] = jnp.where(better, bi, midx[...])

    @pl.when(k == N_K - 1)
    def _emit():
        o_ref[0] = midx[...]


def kernel(x):
    out = pl.pallas_call(
        _argmin_body,
        grid=(N_BATCH, N_K),
        in_specs=[pl.BlockSpec((1, ROW_BLK, N_COL), lambda b, k: (b, k, 0))],
        out_specs=pl.BlockSpec((1, 1, N_COL), lambda b, k: (b, 0, 0)),
        out_shape=jax.ShapeDtypeStruct((N_BATCH, 1, N_COL), jnp.int32),
        scratch_shapes=[
            pltpu.VMEM((1, N_COL), jnp.float32),
            pltpu.VMEM((1, N_COL), jnp.int32),
        ],
    )(x)
    return out.reshape(N_BATCH, N_COL).astype(jnp.int64)


# speedup vs baseline: 2.3306x; 1.0021x over previous
import jax
import jax.numpy as jnp
from jax.experimental import pallas as pl
from jax.experimental.pallas import tpu as pltpu

ROW_BLK = 2048
N_ROW = 4096
N_COL = 2048
N_BATCH = 4
N_K = N_ROW // ROW_BLK


def _argmin_body(x_ref, o_ref, mval, midx):
    k = pl.program_id(1)
    SUB = 256
    bm = None
    bi = None
    for j in range(ROW_BLK // SUB):
        sub = x_ref[0, pl.ds(j * SUB, SUB), :]
        m = jnp.min(sub, axis=0, keepdims=True)
        rows = (
            jax.lax.broadcasted_iota(jnp.int32, (SUB, N_COL), 0)
            + (k * ROW_BLK + j * SUB)
        )
        im = jnp.min(
            jnp.where(sub == m, rows, jnp.int32(2**30)), axis=0, keepdims=True
        )
        if bm is None:
            bm, bi = m, im
        else:
            better = m < bm
            bm = jnp.where(better, m, bm)
            bi = jnp.where(better, im, bi)

    @pl.when(k == 0)
    def _init():
        mval[...] = bm
        midx[...] = bi

    @pl.when(k > 0)
    def _merge():
        better = bm < mval[...]
        mval[...] = jnp.where(better, bm, mval[...])
        midx[...] = jnp.where(better, bi, midx[...])

    @pl.when(k == N_K - 1)
    def _emit():
        o_ref[0] = midx[...]


def kernel(x):
    out = pl.pallas_call(
        _argmin_body,
        grid=(N_BATCH, N_K),
        in_specs=[pl.BlockSpec((1, ROW_BLK, N_COL), lambda b, k: (b, k, 0))],
        out_specs=pl.BlockSpec((1, 1, N_COL), lambda b, k: (b, 0, 0)),
        out_shape=jax.ShapeDtypeStruct((N_BATCH, 1, N_COL), jnp.int32),
        scratch_shapes=[
            pltpu.VMEM((1, N_COL), jnp.float32),
            pltpu.VMEM((1, N_COL), jnp.int32),
        ],
    )(x)
    return out.reshape(N_BATCH, N_COL).astype(jnp.int64)
